# baseline (speedup 1.0000x reference)
import functools

import jax
import jax.numpy as jnp
from jax import lax
from jax.experimental import pallas as pl
from jax.experimental.pallas import tpu as pltpu
from jax.experimental.pallas import tpu_sc as plsc

_BR = 256
_BC = 1024


def _vq_dist_kernel(x_ref, w_ref, x2_ref, w2_ref, idx_ref, loss_ref):
    i = pl.program_id(0)
    kk = w_ref.shape[0]
    nchunks = kk // _BC

    @pl.when(i == 0)
    def _():
        loss_ref[...] = jnp.zeros((1, 1), jnp.float32)

    x = x_ref[...]
    x2 = x2_ref[...]

    init = (jnp.full((_BR,), jnp.inf, jnp.float32),
            jnp.zeros((_BR,), jnp.int32),
            jnp.full((_BR,), jnp.inf, jnp.float32))

    def body(c, carry):
        run_min, run_idx, run_d2 = carry
        w = w_ref[pl.ds(c * _BC, _BC), :]
        w2 = w2_ref[:, pl.ds(c * _BC, _BC)]
        xw = lax.dot_general(x, w, (((1,), (1,)), ((), ())),
                             preferred_element_type=jnp.float32)
        d2 = x2 - 2.0 * xw + w2
        dist = jnp.sqrt(jnp.maximum(d2, 0.0))
        lmin = jnp.min(dist, axis=1)
        col = lax.broadcasted_iota(jnp.int32, (_BR, _BC), 1) + c * _BC
        lidx = jnp.min(jnp.where(dist == lmin[:, None], col, jnp.int32(2**30)),
                       axis=1)
        ld2 = jnp.min(d2, axis=1)
        better = lmin < run_min
        return (jnp.where(better, lmin, run_min),
                jnp.where(better, lidx, run_idx),
                jnp.minimum(ld2, run_d2))

    _, run_idx, run_d2 = lax.fori_loop(0, nchunks, body, init)
    idx_ref[0, 0, :] = run_idx
    loss_ref[...] += jnp.sum(run_d2).reshape(1, 1)


def _dist_argmin(flat_x, W, x2, w2):
    n, d = flat_x.shape
    kk = W.shape[0]
    grid = (n // _BR,)
    idx3, loss_acc = pl.pallas_call(
        _vq_dist_kernel,
        grid=grid,
        in_specs=[
            pl.BlockSpec((_BR, d), lambda i: (i, 0)),
            pl.BlockSpec((kk, d), lambda i: (0, 0)),
            pl.BlockSpec((_BR, 1), lambda i: (i, 0)),
            pl.BlockSpec((1, kk), lambda i: (0, 0)),
        ],
        out_specs=[
            pl.BlockSpec((1, 1, _BR), lambda i: (i, 0, 0)),
            pl.BlockSpec((1, 1), lambda i: (0, 0)),
        ],
        out_shape=[
            jax.ShapeDtypeStruct((grid[0], 1, _BR), jnp.int32),
            jax.ShapeDtypeStruct((1, 1), jnp.float32),
        ],
    )(flat_x, W, x2, w2)
    return idx3.reshape(n), loss_acc[0, 0]


def _sc_gather(W, idx):
    info = plsc.get_sparse_core_info()
    nc, ns = info.num_cores, info.num_subcores
    nw = nc * ns
    b, d = idx.shape[0], W.shape[1]
    b_per_w = b // nw
    ch = 96
    nch = b_per_w // ch
    mesh = plsc.VectorSubcoreMesh(core_axis_name="c", subcore_axis_name="s")

    @functools.partial(
        pl.kernel, mesh=mesh,
        out_type=jax.ShapeDtypeStruct((b, d), jnp.float32),
        scratch_types=[
            pltpu.VMEM((b_per_w,), jnp.int32),
            pltpu.VMEM((ch, d), jnp.float32),
            pltpu.VMEM((ch, d), jnp.float32),
            pltpu.SemaphoreType.DMA,
            pltpu.SemaphoreType.DMA,
        ],
    )
    def gk(table_hbm, idx_hbm, out_hbm, idx_v, buf0, buf1, sem0, sem1):
        wid = lax.axis_index("s") * nc + lax.axis_index("c")
        base = wid * b_per_w
        pltpu.sync_copy(idx_hbm.at[pl.ds(base, b_per_w)], idx_v)
        bufs = (buf0, buf1)
        sems = (sem0, sem1)
        copies = []
        for c in range(nch):
            copies.append(pltpu.async_copy(
                table_hbm.at[idx_v.at[pl.ds(c * ch, ch)]],
                bufs[c % 2], sems[c % 2]))
            if c >= 1:
                copies[c - 1].wait()
                pltpu.sync_copy(bufs[(c - 1) % 2],
                                out_hbm.at[pl.ds(base + (c - 1) * ch, ch)])
        copies[nch - 1].wait()
        pltpu.sync_copy(bufs[(nch - 1) % 2],
                        out_hbm.at[pl.ds(base + (nch - 1) * ch, ch)])

    return gk(W, idx)


def kernel(x, W):
    b, s, d = x.shape
    flat_x = x.reshape(b * s, d)
    x2 = jnp.sum(flat_x * flat_x, axis=1, keepdims=True)
    w2 = jnp.sum(W * W, axis=1)[None, :]
    idx, loss_acc = _dist_argmin(flat_x, W, x2, w2)
    quantized = _sc_gather(W, idx).reshape(b, s, d)
    loss = loss_acc * jnp.float32(1.25) / jnp.float32(flat_x.size)
    return quantized, loss
